# Initial kernel scaffold; baseline (speedup 1.0000x reference)
import jax
import jax.numpy as jnp
from jax.experimental import pallas as pl


def kernel(x, edge_index, graph_ids, W_pre, b_pre, W1, b1, W2, b2, W3, b3, Wm1, bm1, Wm2, bm2, Wm3, bm3):
    raise NotImplementedError("write your pallas kernel here")


# speedup vs baseline: 3.5517x; 3.5517x over previous
import functools

import jax
import jax.numpy as jnp
from jax import lax
from jax.experimental import pallas as pl
from jax.experimental.pallas import tpu as pltpu
from jax.experimental.pallas import tpu_sc as plsc

_N = 10000
_D = 128
_D2 = 64
_H = 128
_G = 128
_E = 320000
_M1, _M2 = 256, 128

_NP = 10240
_NW = 32
_C = 128
_NB = 79
_EPAD = _NW * _NB * _C
_RPT = _NP // 16

_mesh = plsc.VectorSubcoreMesh(core_axis_name="c", subcore_axis_name="s")

_half = jax.ShapeDtypeStruct((_NP, _D2), jnp.float32)


@functools.partial(
    pl.kernel,
    out_type=(_half, _half, _half, _half),
    mesh=_mesh,
    scratch_types=[
        pltpu.VMEM((_NB, _C), jnp.int32),
        pltpu.VMEM((_NB, _C), jnp.int32),
        pltpu.VMEM((_C, _D2), jnp.float32),
        pltpu.VMEM((_C, _D2), jnp.float32),
        pltpu.VMEM_SHARED((_NP, _D2), jnp.float32),
        pltpu.SemaphoreType.DMA,
    ],
    compiler_params=pltpu.CompilerParams(use_tc_tiling_on_sc=False),
)
def _sc_agg(h_lo, h_hi, src_hbm, dst_hbm, p0l, p1l, p0h, p1h,
            src_v, dst_v, rows_v, zbuf, acc, sem):
    c = lax.axis_index("c")
    s = lax.axis_index("s")
    wid = s * 2 + c
    row0 = pl.multiple_of(s * _RPT, _RPT)

    pltpu.sync_copy(src_hbm.at[wid], src_v)
    pltpu.sync_copy(dst_hbm.at[wid], dst_v)

    def zrow(i, carry):
        for j in range(_D2 // 16):
            zbuf[i, pl.ds(j * 16, 16)] = jnp.zeros((16,), jnp.float32)
        return carry

    lax.fori_loop(0, _C, zrow, 0)

    for h_hbm, out0, out1 in ((h_lo, p0l, p1l), (h_hi, p0h, p1h)):
        @pl.when(c == 0)
        def _():
            pltpu.sync_copy(h_hbm.at[pl.ds(row0, _RPT)], acc.at[pl.ds(row0, _RPT)])

        @pl.when(c != 0)
        def _():
            for k in range(_RPT // _C):
                pltpu.sync_copy(zbuf, acc.at[pl.ds(row0 + k * _C, _C)])

        plsc.subcore_barrier()

        def edge_chunk(j, carry):
            pltpu.async_copy(h_hbm.at[src_v.at[j]], rows_v, sem).wait()
            pltpu.sync_copy(rows_v, acc.at[dst_v.at[j]], add=True)
            return carry

        lax.fori_loop(0, _NB, edge_chunk, 0)

        plsc.subcore_barrier()

        @pl.when(c == 0)
        def _():
            pltpu.sync_copy(acc.at[pl.ds(row0, _RPT)], out0.at[pl.ds(row0, _RPT)])

        @pl.when(c != 0)
        def _():
            pltpu.sync_copy(acc.at[pl.ds(row0, _RPT)], out1.at[pl.ds(row0, _RPT)])


def _dense_body(p0l_ref, p1l_ref, p0h_ref, p1h_ref, w_ref, b_ref, olo_ref, ohi_ref):
    h = jnp.concatenate(
        [p0l_ref[...] + p1l_ref[...], p0h_ref[...] + p1h_ref[...]], axis=1)
    z = jnp.dot(h, w_ref[...], preferred_element_type=jnp.float32) + b_ref[...]
    z = jnp.maximum(z, 0.0)
    olo_ref[...] = z[:, :_D2]
    ohi_ref[...] = z[:, _D2:]


def _dense(p0l, p1l, p0h, p1h, w, b):
    blk = _NP // 8
    half_spec = pl.BlockSpec((blk, _D2), lambda i: (i, 0))
    return pl.pallas_call(
        _dense_body,
        grid=(8,),
        in_specs=[
            half_spec, half_spec, half_spec, half_spec,
            pl.BlockSpec((_D, _H), lambda i: (0, 0)),
            pl.BlockSpec((1, _H), lambda i: (0, 0)),
        ],
        out_specs=[half_spec, half_spec],
        out_shape=[_half, _half],
    )(p0l, p1l, p0h, p1h, w, b)


def _head_body(hlo_ref, hhi_ref, gid_ref, wm1_ref, bm1_ref, wm2_ref, bm2_ref,
               wm3_ref, bm3_ref, o_ref):
    h = jnp.concatenate([hlo_ref[...], hhi_ref[...]], axis=1)
    ids = gid_ref[...]
    onehot = (lax.broadcasted_iota(jnp.int32, (_G, _N), 0) == ids).astype(jnp.float32)
    sums = jnp.dot(onehot, h, preferred_element_type=jnp.float32)
    counts = jnp.sum(onehot, axis=1, keepdims=True)
    mean = sums / jnp.maximum(counts, 1.0)
    z = jnp.dot(mean, wm1_ref[...], preferred_element_type=jnp.float32) + bm1_ref[...]
    z = jnp.where(z > 0, z, 0.01 * z)
    z = jnp.dot(z, wm2_ref[...], preferred_element_type=jnp.float32) + bm2_ref[...]
    z = jnp.where(z > 0, z, 0.01 * z)
    o_ref[...] = jnp.dot(z, wm3_ref[...], preferred_element_type=jnp.float32) + bm3_ref[...]


def _head(hlo, hhi, gid, wm1, bm1, wm2, bm2, wm3, bm3):
    return pl.pallas_call(
        _head_body,
        out_shape=jax.ShapeDtypeStruct((_G, _H), jnp.float32),
    )(hlo, hhi, gid, wm1, bm1, wm2, bm2, wm3, bm3)


def kernel(x, edge_index, graph_ids, W_pre, b_pre, W1, b1, W2, b2, W3, b3,
           Wm1, bm1, Wm2, bm2, Wm3, bm3):
    src = edge_index[0].astype(jnp.int32)
    dst = edge_index[1].astype(jnp.int32)
    pad = _EPAD - _E
    src = jnp.concatenate([src, jnp.zeros((pad,), jnp.int32)]).reshape(_NW, _NB, _C)
    dst = jnp.concatenate([dst, jnp.full((pad,), _N, jnp.int32)]).reshape(_NW, _NB, _C)

    hlo = jnp.pad(x[:, :_D2], ((0, _NP - _N), (0, 0)))
    hhi = jnp.pad(x[:, _D2:], ((0, _NP - _N), (0, 0)))
    for w, b in ((W_pre, b_pre), (W1, b1), (W2, b2), (W3, b3)):
        p0l, p1l, p0h, p1h = _sc_agg(hlo, hhi, src, dst)
        hlo, hhi = _dense(p0l, p1l, p0h, p1h, w, b.reshape(1, _H))

    gid = graph_ids.astype(jnp.int32).reshape(1, _N)
    wm3 = jnp.pad(Wm3, ((0, 0), (0, _H - 1)))
    bm3 = jnp.pad(bm3, (0, _H - 1)).reshape(1, _H)
    out = _head(hlo[:_N], hhi[:_N], gid, Wm1, bm1.reshape(1, _M1),
                Wm2, bm2.reshape(1, _M2), wm3, bm3)
    return out[:, 0]
